# baseline (speedup 1.0000x reference)
import functools

import jax
import jax.numpy as jnp
from jax import lax
from jax.experimental import pallas as pl
from jax.experimental.pallas import tpu as pltpu
from jax.experimental.pallas import tpu_sc as plsc

B, S, D = 4, 2048, 1024
BLOCK_ROWS = 2048

NC, NS, L = 2, 16, 16
NW = NC * NS
RW = S // NW
CHUNK = 16
QPB = RW // CHUNK


def _add_body(x_ref, e_ref, o_ref):
    o_ref[...] = x_ref[...] + e_ref[...]


def _tc_part(x, E, b0):
    nb = B - b0
    grid = (S // BLOCK_ROWS, nb)
    return pl.pallas_call(
        _add_body,
        grid=grid,
        in_specs=[
            pl.BlockSpec((1, BLOCK_ROWS, D), lambda i, b: (b + b0, i, 0)),
            pl.BlockSpec((BLOCK_ROWS, D), lambda i, b: (i, 0)),
        ],
        out_specs=pl.BlockSpec((1, BLOCK_ROWS, D), lambda i, b: (b, i, 0)),
        out_shape=jax.ShapeDtypeStruct((nb, S, D), x.dtype),
    )(x, E)


@functools.cache
def _build_sc_part(nb):
    mesh = plsc.VectorSubcoreMesh(core_axis_name="c", subcore_axis_name="s")
    nchunk = nb * QPB

    @functools.partial(
        pl.kernel,
        mesh=mesh,
        out_type=jax.ShapeDtypeStruct((nb, S, D), jnp.float32),
        scratch_types=[
            pltpu.VMEM((RW, D), jnp.float32),
            pltpu.VMEM((CHUNK, D), jnp.float32),
            pltpu.VMEM((CHUNK, D), jnp.float32),
            pltpu.VMEM((CHUNK, D), jnp.float32),
            pltpu.SemaphoreType.DMA,
            pltpu.SemaphoreType.DMA,
            pltpu.SemaphoreType.DMA,
            pltpu.SemaphoreType.DMA,
            pltpu.SemaphoreType.DMA,
            pltpu.SemaphoreType.DMA,
            pltpu.SemaphoreType.DMA,
        ],
    )
    def sc_add(x_hbm, e_hbm, o_hbm, ebuf, xb0, xb1, xb2,
               esem, is0, is1, is2, os0, os1, os2):
        wid = lax.axis_index("s") * NC + lax.axis_index("c")
        e_base = wid * RW
        xbufs = (xb0, xb1, xb2)
        isems = (is0, is1, is2)
        osems = (os0, os1, os2)

        def src(c):
            b, q = divmod(c, QPB)
            return x_hbm.at[b, pl.ds(e_base + q * CHUNK, CHUNK)]

        def dst(c):
            b, q = divmod(c, QPB)
            return o_hbm.at[b, pl.ds(e_base + q * CHUNK, CHUNK)]

        e_src = e_hbm.at[pl.ds(e_base, RW)]
        pltpu.async_copy(e_src, ebuf, esem)
        pltpu.async_copy(src(0), xbufs[0], isems[0])
        pltpu.make_async_copy(e_src, ebuf, esem).wait()

        for c in range(nchunk):
            k = c % 3
            xb = xbufs[k]
            if c >= 2:
                kk = (c - 2) % 3
                pltpu.make_async_copy(xbufs[kk], dst(c - 2), osems[kk]).wait()
            if c + 1 < nchunk:
                kn = (c + 1) % 3
                pltpu.async_copy(src(c + 1), xbufs[kn], isems[kn])
            pltpu.make_async_copy(src(c), xb, isems[k]).wait()

            qbase = (c % QPB) * CHUNK
            UNROLL = 8

            @plsc.parallel_loop(0, CHUNK * (D // (L * UNROLL)))
            def _(j):
                r = j // (D // (L * UNROLL))
                cb = (j % (D // (L * UNROLL))) * (L * UNROLL)
                for u in range(UNROLL):
                    sl = pl.ds(cb + u * L, L)
                    plsc.addupdate(xb.at[r, sl], ebuf[qbase + r, sl])

            pltpu.async_copy(xb, dst(c), osems[k])

        for c in range(max(nchunk - 2, 0), nchunk):
            k = c % 3
            pltpu.make_async_copy(xbufs[k], dst(c), osems[k]).wait()

    return sc_add


NB_SC = 1


def kernel(x, E):
    sc_out = _build_sc_part(NB_SC)(x, E)
    tc_out = _tc_part(x, E, NB_SC)
    return jnp.concatenate([sc_out, tc_out], axis=0)


# speedup vs baseline: 2.5036x; 2.5036x over previous
import functools

import jax
import jax.numpy as jnp
from jax import lax
from jax.experimental import pallas as pl
from jax.experimental.pallas import tpu as pltpu
from jax.experimental.pallas import tpu_sc as plsc

B, S, D = 4, 2048, 1024
BLOCK_ROWS = 2048

NC, NS, L = 2, 16, 16
NW = NC * NS
RW = S // NW
CHUNK = 16
QPB = RW // CHUNK


def _add_body(x_ref, e_ref, o_ref):
    o_ref[...] = x_ref[...] + e_ref[...]


BLOCK_D = 512


def _tc_part(x, E, b0):
    nb = B - b0
    grid = (D // BLOCK_D, nb)
    return pl.pallas_call(
        _add_body,
        grid=grid,
        in_specs=[
            pl.BlockSpec((1, S, BLOCK_D), lambda d, b: (b + b0, 0, d)),
            pl.BlockSpec((S, BLOCK_D), lambda d, b: (0, d)),
        ],
        out_specs=pl.BlockSpec((1, S, BLOCK_D), lambda d, b: (b, 0, d)),
        out_shape=jax.ShapeDtypeStruct((nb, S, D), x.dtype),
    )(x, E)


@functools.cache
def _build_sc_part(nb):
    mesh = plsc.VectorSubcoreMesh(core_axis_name="c", subcore_axis_name="s")
    nchunk = nb * QPB

    @functools.partial(
        pl.kernel,
        mesh=mesh,
        out_type=jax.ShapeDtypeStruct((nb, S, D), jnp.float32),
        scratch_types=[
            pltpu.VMEM((RW, D), jnp.float32),
            pltpu.VMEM((CHUNK, D), jnp.float32),
            pltpu.VMEM((CHUNK, D), jnp.float32),
            pltpu.VMEM((CHUNK, D), jnp.float32),
            pltpu.SemaphoreType.DMA,
            pltpu.SemaphoreType.DMA,
            pltpu.SemaphoreType.DMA,
            pltpu.SemaphoreType.DMA,
            pltpu.SemaphoreType.DMA,
            pltpu.SemaphoreType.DMA,
            pltpu.SemaphoreType.DMA,
        ],
    )
    def sc_add(x_hbm, e_hbm, o_hbm, ebuf, xb0, xb1, xb2,
               esem, is0, is1, is2, os0, os1, os2):
        wid = lax.axis_index("s") * NC + lax.axis_index("c")
        e_base = wid * RW
        xbufs = (xb0, xb1, xb2)
        isems = (is0, is1, is2)
        osems = (os0, os1, os2)

        def src(c):
            b, q = divmod(c, QPB)
            return x_hbm.at[b, pl.ds(e_base + q * CHUNK, CHUNK)]

        def dst(c):
            b, q = divmod(c, QPB)
            return o_hbm.at[b, pl.ds(e_base + q * CHUNK, CHUNK)]

        e_src = e_hbm.at[pl.ds(e_base, RW)]
        pltpu.async_copy(e_src, ebuf, esem)
        pltpu.async_copy(src(0), xbufs[0], isems[0])
        pltpu.make_async_copy(e_src, ebuf, esem).wait()

        for c in range(nchunk):
            k = c % 3
            xb = xbufs[k]
            if c >= 2:
                kk = (c - 2) % 3
                pltpu.make_async_copy(xbufs[kk], dst(c - 2), osems[kk]).wait()
            if c + 1 < nchunk:
                kn = (c + 1) % 3
                pltpu.async_copy(src(c + 1), xbufs[kn], isems[kn])
            pltpu.make_async_copy(src(c), xb, isems[k]).wait()

            qbase = (c % QPB) * CHUNK
            UNROLL = 8

            @plsc.parallel_loop(0, CHUNK * (D // (L * UNROLL)))
            def _(j):
                r = j // (D // (L * UNROLL))
                cb = (j % (D // (L * UNROLL))) * (L * UNROLL)
                for u in range(UNROLL):
                    sl = pl.ds(cb + u * L, L)
                    plsc.addupdate(xb.at[r, sl], ebuf[qbase + r, sl])

            pltpu.async_copy(xb, dst(c), osems[k])

        for c in range(max(nchunk - 2, 0), nchunk):
            k = c % 3
            pltpu.make_async_copy(xbufs[k], dst(c), osems[k]).wait()

    return sc_add


def kernel(x, E):
    return _tc_part(x, E, 0)


# speedup vs baseline: 2.7631x; 1.1036x over previous
import functools

import jax
import jax.numpy as jnp
from jax import lax
from jax.experimental import pallas as pl
from jax.experimental.pallas import tpu as pltpu
from jax.experimental.pallas import tpu_sc as plsc

B, S, D = 4, 2048, 1024
BLOCK_ROWS = 2048

NC, NS, L = 2, 16, 16
NW = NC * NS
RW = S // NW
CHUNK = 16
QPB = RW // CHUNK


def _add_body(x_ref, e_ref, o_ref):
    o_ref[...] = x_ref[...] + e_ref[...]


def _tc_part(x, E, b0):
    nb = B - b0
    grid = (S // BLOCK_ROWS, nb)
    return pl.pallas_call(
        _add_body,
        grid=grid,
        in_specs=[
            pl.BlockSpec((1, BLOCK_ROWS, D), lambda i, b: (b + b0, i, 0)),
            pl.BlockSpec((BLOCK_ROWS, D), lambda i, b: (i, 0)),
        ],
        out_specs=pl.BlockSpec((1, BLOCK_ROWS, D), lambda i, b: (b, i, 0)),
        out_shape=jax.ShapeDtypeStruct((nb, S, D), x.dtype),
    )(x, E)


@functools.cache
def _build_sc_part(nb):
    mesh = plsc.VectorSubcoreMesh(core_axis_name="c", subcore_axis_name="s")
    nchunk = nb * QPB

    @functools.partial(
        pl.kernel,
        mesh=mesh,
        out_type=jax.ShapeDtypeStruct((nb, S, D), jnp.float32),
        scratch_types=[
            pltpu.VMEM((RW, D), jnp.float32),
            pltpu.VMEM((CHUNK, D), jnp.float32),
            pltpu.VMEM((CHUNK, D), jnp.float32),
            pltpu.VMEM((CHUNK, D), jnp.float32),
            pltpu.SemaphoreType.DMA,
            pltpu.SemaphoreType.DMA,
            pltpu.SemaphoreType.DMA,
            pltpu.SemaphoreType.DMA,
            pltpu.SemaphoreType.DMA,
            pltpu.SemaphoreType.DMA,
            pltpu.SemaphoreType.DMA,
        ],
    )
    def sc_add(x_hbm, e_hbm, o_hbm, ebuf, xb0, xb1, xb2,
               esem, is0, is1, is2, os0, os1, os2):
        wid = lax.axis_index("s") * NC + lax.axis_index("c")
        e_base = wid * RW
        xbufs = (xb0, xb1, xb2)
        isems = (is0, is1, is2)
        osems = (os0, os1, os2)

        def src(c):
            b, q = divmod(c, QPB)
            return x_hbm.at[b, pl.ds(e_base + q * CHUNK, CHUNK)]

        def dst(c):
            b, q = divmod(c, QPB)
            return o_hbm.at[b, pl.ds(e_base + q * CHUNK, CHUNK)]

        e_src = e_hbm.at[pl.ds(e_base, RW)]
        pltpu.async_copy(e_src, ebuf, esem)
        pltpu.async_copy(src(0), xbufs[0], isems[0])
        pltpu.make_async_copy(e_src, ebuf, esem).wait()

        for c in range(nchunk):
            k = c % 3
            xb = xbufs[k]
            if c >= 2:
                kk = (c - 2) % 3
                pltpu.make_async_copy(xbufs[kk], dst(c - 2), osems[kk]).wait()
            if c + 1 < nchunk:
                kn = (c + 1) % 3
                pltpu.async_copy(src(c + 1), xbufs[kn], isems[kn])
            pltpu.make_async_copy(src(c), xb, isems[k]).wait()

            qbase = (c % QPB) * CHUNK
            UNROLL = 8

            @plsc.parallel_loop(0, CHUNK * (D // (L * UNROLL)))
            def _(j):
                r = j // (D // (L * UNROLL))
                cb = (j % (D // (L * UNROLL))) * (L * UNROLL)
                for u in range(UNROLL):
                    sl = pl.ds(cb + u * L, L)
                    plsc.addupdate(xb.at[r, sl], ebuf[qbase + r, sl])

            pltpu.async_copy(xb, dst(c), osems[k])

        for c in range(max(nchunk - 2, 0), nchunk):
            k = c % 3
            pltpu.make_async_copy(xbufs[k], dst(c), osems[k]).wait()

    return sc_add


def kernel(x, E):
    return _tc_part(x, E, 0)
